# baseline (speedup 1.0000x reference)
import functools

import jax
import jax.numpy as jnp
from jax import lax
from jax.experimental import pallas as pl
from jax.experimental.pallas import tpu as pltpu
from jax.experimental.pallas import tpu_sc as plsc

N = 10000
E = 320000
G = 64
D = 128

NC = 2
NS = 16
NW = NC * NS
CH = 128
NBUF = 2
KT = 160
K0 = 112
K1 = KT - K0
KM = max(K0, K1)
KD = KT // NC
E_PAD = NS * KT * CH
NPAD = 10112
RPS = NPAD // NS
RTL = RPS - 4 * CH

NB = 10
R = N // NB

_MESH = plsc.VectorSubcoreMesh(core_axis_name="c", subcore_axis_name="s")


@functools.partial(
    pl.kernel,
    out_type=jax.ShapeDtypeStruct((NC, NPAD, D), jnp.float32),
    mesh=_MESH,
    scratch_types=[
        pltpu.VMEM((KD, CH), jnp.int32),
        pltpu.VMEM((CH, D), jnp.float32),
        pltpu.VMEM_SHARED((NPAD, D), jnp.float32),
        pltpu.SemaphoreType.DMA((NBUF,)),
    ],
)
def _deg_pass(dst_hbm, out_hbm, dstv, ones, acc, ssem):
    c = lax.axis_index("c")
    s = lax.axis_index("s")

    @pl.loop(0, CH)
    def _zr(r):
        @pl.loop(0, D, step=16)
        def _zc(cc):
            ones[r, pl.ds(cc, 16)] = jnp.zeros((16,), jnp.float32)

    @pl.loop(0, RPS - RTL, step=CH)
    def _za(r0):
        pltpu.sync_copy(ones, acc.at[pl.ds(s * RPS + r0, CH)])

    pltpu.sync_copy(ones.at[pl.ds(0, RTL)],
                    acc.at[pl.ds(s * RPS + RPS - RTL, RTL)])

    @pl.loop(0, CH)
    def _or(r):
        @pl.loop(0, D, step=16)
        def _oc(cc):
            ones[r, pl.ds(cc, 16)] = jnp.ones((16,), jnp.float32)

    plsc.subcore_barrier()
    pltpu.sync_copy(dst_hbm.at[s, pl.ds(c * KD, KD)], dstv)

    @pl.loop(0, KD, step=NBUF)
    def _chunk(j0):
        scs = [
            pltpu.async_copy(ones, acc.at[dstv.at[j0 + b]], ssem.at[b],
                             add=True)
            for b in range(NBUF)
        ]
        for cp in scs:
            cp.wait()

    plsc.subcore_barrier()
    pltpu.sync_copy(acc.at[pl.ds(s * RPS, RPS)],
                    out_hbm.at[c, pl.ds(s * RPS, RPS)])


@functools.partial(
    pl.kernel,
    out_type=jax.ShapeDtypeStruct((NC, NPAD, D), jnp.float32),
    mesh=_MESH,
    scratch_types=[
        pltpu.VMEM((KM, CH), jnp.int32),
    ]
    + [pltpu.VMEM((CH,), jnp.int32) for _ in range(NBUF)]
    + [pltpu.VMEM((CH, D), jnp.float32) for _ in range(NBUF)]
    + [
        pltpu.VMEM_SHARED((NPAD, D), jnp.float32),
        pltpu.SemaphoreType.DMA((NBUF,)),
        pltpu.SemaphoreType.DMA((NBUF,)),
        pltpu.SemaphoreType.DMA((NBUF,)),
    ],
)
def _edge_pass(g_hbm, src_hbm, dst_hbm, out_hbm, srcv, d0, d1, r0, r1,
               acc, gsem, ssem, dsem):
    c = lax.axis_index("c")
    s = lax.axis_index("s")
    rows = [r0, r1]
    dbufs = [d0, d1]

    @pl.loop(0, CH)
    def _zr(r):
        @pl.loop(0, D, step=16)
        def _zc(cc):
            r0[r, pl.ds(cc, 16)] = jnp.zeros((16,), jnp.float32)

    @pl.loop(0, RPS - RTL, step=CH)
    def _za(rr):
        pltpu.sync_copy(r0, acc.at[pl.ds(s * RPS + rr, CH)])

    pltpu.sync_copy(r0.at[pl.ds(0, RTL)],
                    acc.at[pl.ds(s * RPS + RPS - RTL, RTL)])

    plsc.subcore_barrier()

    def _run_chunks(base, kcount):
        pltpu.sync_copy(src_hbm.at[s, pl.ds(base, kcount)],
                        srcv.at[pl.ds(0, kcount)])

        @pl.loop(0, kcount, step=NBUF)
        def _chunk(j0):
            ds = [
                pltpu.async_copy(dst_hbm.at[s, base + j0 + b], dbufs[b],
                                 dsem.at[b])
                for b in range(NBUF)
            ]
            gs = [
                pltpu.async_copy(g_hbm.at[srcv.at[j0 + b]], rows[b],
                                 gsem.at[b])
                for b in range(NBUF)
            ]
            scs = []
            for b in range(NBUF):
                ds[b].wait()
                gs[b].wait()
                scs.append(
                    pltpu.async_copy(rows[b], acc.at[dbufs[b]],
                                     ssem.at[b], add=True))
            for cp in scs:
                cp.wait()

    @pl.when(c == 0)
    def _c0():
        _run_chunks(0, K0)

    @pl.when(c == 1)
    def _c1():
        _run_chunks(K0, K1)

    plsc.subcore_barrier()
    pltpu.sync_copy(acc.at[pl.ds(s * RPS, RPS)],
                    out_hbm.at[c, pl.ds(s * RPS, RPS)])


def _tc_mm_body(x_ref, w_ref, h_ref):
    h_ref[...] = jnp.dot(x_ref[...], w_ref[...],
                         preferred_element_type=jnp.float32)


_tc_mm = pl.pallas_call(
    _tc_mm_body,
    grid=(NB,),
    in_specs=[
        pl.BlockSpec((R, D), lambda i: (i, 0)),
        pl.BlockSpec((D, D), lambda i: (0, 0)),
    ],
    out_specs=pl.BlockSpec((R, D), lambda i: (i, 0)),
    out_shape=jax.ShapeDtypeStruct((N, D), jnp.float32),
)


def _tc1_body(deg_ref, h_ref, g_ref, dinv_ref):
    deg = deg_ref[0, :, 0:1] + deg_ref[1, :, 0:1] + 1.0
    dinv = lax.rsqrt(deg)
    g_ref[...] = h_ref[...] * dinv
    dinv_ref[...] = dinv


_tc1 = pl.pallas_call(
    _tc1_body,
    grid=(NB,),
    in_specs=[
        pl.BlockSpec((2, R, D), lambda i: (0, i, 0)),
        pl.BlockSpec((R, D), lambda i: (i, 0)),
    ],
    out_specs=[
        pl.BlockSpec((R, D), lambda i: (i, 0)),
        pl.BlockSpec((R, 1), lambda i: (i, 0)),
    ],
    out_shape=[
        jax.ShapeDtypeStruct((N, D), jnp.float32),
        jax.ShapeDtypeStruct((N, 1), jnp.float32),
    ],
)


def _tc_mid_body(sp_ref, g_ref, dinv_ref, b_ref, w_ref, gout_ref):
    ssum = sp_ref[0] + sp_ref[1]
    dinv = dinv_ref[...]
    f = jnp.maximum(dinv * (ssum + g_ref[...]) + b_ref[...], 0.0)
    gout_ref[...] = jnp.dot(
        f, w_ref[...], preferred_element_type=jnp.float32) * dinv


_tc_mid = pl.pallas_call(
    _tc_mid_body,
    grid=(NB,),
    in_specs=[
        pl.BlockSpec((2, R, D), lambda i: (0, i, 0)),
        pl.BlockSpec((R, D), lambda i: (i, 0)),
        pl.BlockSpec((R, 1), lambda i: (i, 0)),
        pl.BlockSpec((1, D), lambda i: (0, 0)),
        pl.BlockSpec((D, D), lambda i: (0, 0)),
    ],
    out_specs=pl.BlockSpec((R, D), lambda i: (i, 0)),
    out_shape=jax.ShapeDtypeStruct((N, D), jnp.float32),
)


def _tc_final_body(sp_ref, g_ref, dinv_ref, b_ref, batch_ref, wl_ref, bl_ref,
                   out_ref):
    ssum = sp_ref[0, :N, :] + sp_ref[1, :N, :]
    h = dinv_ref[...] * (ssum + g_ref[...]) + b_ref[...]
    gid = lax.broadcasted_iota(jnp.int32, (G, N), 0)
    oht = (batch_ref[...] == gid).astype(jnp.float32)
    sums = jnp.dot(oht, h, preferred_element_type=jnp.float32)
    counts = jnp.dot(oht, jnp.ones((N, 1), jnp.float32),
                     preferred_element_type=jnp.float32)
    pooled = sums / jnp.maximum(counts, 1.0)
    out_ref[...] = jnp.dot(
        pooled, wl_ref[...], preferred_element_type=jnp.float32) + bl_ref[...]


_tc_final = pl.pallas_call(
    _tc_final_body,
    out_shape=jax.ShapeDtypeStruct((G, 1), jnp.float32),
)


def kernel(x, edge_index, batch, W1, b1, W2, b2, W3, b3, Wl, bl):
    src = edge_index[0].astype(jnp.int32)
    dst = edge_index[1].astype(jnp.int32)
    pad = E_PAD - E
    src_r = jnp.pad(src, (0, pad)).reshape(NS, KT, CH)
    dst_r = jnp.pad(dst, (0, pad), constant_values=N).reshape(NS, KT, CH)
    batch_row = batch.astype(jnp.int32).reshape(1, N)
    b1r = b1.reshape(1, D)
    b2r = b2.reshape(1, D)
    b3r = b3.reshape(1, D)
    blr = bl.reshape(1, 1)

    degp = _deg_pass(dst_r)
    h1 = _tc_mm(x, W1)
    g1, dinv = _tc1(degp, h1)
    s1 = _edge_pass(g1, src_r, dst_r)
    g2 = _tc_mid(s1, g1, dinv, b1r, W2)
    s2 = _edge_pass(g2, src_r, dst_r)
    g3 = _tc_mid(s2, g2, dinv, b2r, W3)
    s3 = _edge_pass(g3, src_r, dst_r)
    return _tc_final(s3, g3, dinv, b3r, batch_row, Wl, blr)


# speedup vs baseline: 1.0549x; 1.0549x over previous
import functools

import jax
import jax.numpy as jnp
from jax import lax
from jax.experimental import pallas as pl
from jax.experimental.pallas import tpu as pltpu
from jax.experimental.pallas import tpu_sc as plsc

N = 10000
E = 320000
G = 64
D = 128

NC = 2
NS = 16
NW = NC * NS
CH = 128
NBUF = 2
KT = 160
K0 = 120
K1 = KT - K0
KM = max(K0, K1)
KD = KT // NC
E_PAD = NS * KT * CH
NPAD = 10112
RPS = NPAD // NS
RTL = RPS - 4 * CH

NB = 10
R = N // NB

_MESH = plsc.VectorSubcoreMesh(core_axis_name="c", subcore_axis_name="s")


@functools.partial(
    pl.kernel,
    out_type=jax.ShapeDtypeStruct((NC, NPAD, D), jnp.float32),
    mesh=_MESH,
    scratch_types=[
        pltpu.VMEM((KD, CH), jnp.int32),
        pltpu.VMEM((CH, D), jnp.float32),
        pltpu.VMEM_SHARED((NPAD, D), jnp.float32),
        pltpu.SemaphoreType.DMA((NBUF,)),
    ],
)
def _deg_pass(dst_hbm, out_hbm, dstv, ones, acc, ssem):
    c = lax.axis_index("c")
    s = lax.axis_index("s")

    @pl.loop(0, CH)
    def _zr(r):
        @pl.loop(0, D, step=16)
        def _zc(cc):
            ones[r, pl.ds(cc, 16)] = jnp.zeros((16,), jnp.float32)

    @pl.loop(0, RPS - RTL, step=CH)
    def _za(r0):
        pltpu.sync_copy(ones, acc.at[pl.ds(s * RPS + r0, CH)])

    pltpu.sync_copy(ones.at[pl.ds(0, RTL)],
                    acc.at[pl.ds(s * RPS + RPS - RTL, RTL)])

    @pl.loop(0, CH)
    def _or(r):
        @pl.loop(0, D, step=16)
        def _oc(cc):
            ones[r, pl.ds(cc, 16)] = jnp.ones((16,), jnp.float32)

    plsc.subcore_barrier()
    pltpu.sync_copy(dst_hbm.at[s, pl.ds(c * KD, KD)], dstv)

    @pl.loop(0, KD, step=NBUF)
    def _chunk(j0):
        scs = [
            pltpu.async_copy(ones, acc.at[dstv.at[j0 + b]], ssem.at[b],
                             add=True)
            for b in range(NBUF)
        ]
        for cp in scs:
            cp.wait()

    plsc.subcore_barrier()
    pltpu.sync_copy(acc.at[pl.ds(s * RPS, RPS)],
                    out_hbm.at[c, pl.ds(s * RPS, RPS)])


@functools.partial(
    pl.kernel,
    out_type=jax.ShapeDtypeStruct((NC, NPAD, D), jnp.float32),
    mesh=_MESH,
    scratch_types=[
        pltpu.VMEM((KM, CH), jnp.int32),
    ]
    + [pltpu.VMEM((CH,), jnp.int32) for _ in range(NBUF)]
    + [pltpu.VMEM((CH, D), jnp.float32) for _ in range(NBUF)]
    + [
        pltpu.VMEM_SHARED((NPAD, D), jnp.float32),
        pltpu.SemaphoreType.DMA((NBUF,)),
        pltpu.SemaphoreType.DMA((NBUF,)),
        pltpu.SemaphoreType.DMA((NBUF,)),
    ],
)
def _edge_pass(g_hbm, src_hbm, dst_hbm, out_hbm, srcv, d0, d1, r0, r1,
               acc, gsem, ssem, dsem):
    c = lax.axis_index("c")
    s = lax.axis_index("s")
    rows = [r0, r1]
    dbufs = [d0, d1]

    @pl.loop(0, CH)
    def _zr(r):
        @pl.loop(0, D, step=16)
        def _zc(cc):
            r0[r, pl.ds(cc, 16)] = jnp.zeros((16,), jnp.float32)

    @pl.loop(0, RPS - RTL, step=CH)
    def _za(rr):
        pltpu.sync_copy(r0, acc.at[pl.ds(s * RPS + rr, CH)])

    pltpu.sync_copy(r0.at[pl.ds(0, RTL)],
                    acc.at[pl.ds(s * RPS + RPS - RTL, RTL)])

    plsc.subcore_barrier()

    def _run_chunks(base, kcount):
        pltpu.sync_copy(src_hbm.at[s, pl.ds(base, kcount)],
                        srcv.at[pl.ds(0, kcount)])

        @pl.loop(0, kcount, step=NBUF)
        def _chunk(j0):
            ds = [
                pltpu.async_copy(dst_hbm.at[s, base + j0 + b], dbufs[b],
                                 dsem.at[b])
                for b in range(NBUF)
            ]
            gs = [
                pltpu.async_copy(g_hbm.at[srcv.at[j0 + b]], rows[b],
                                 gsem.at[b])
                for b in range(NBUF)
            ]
            scs = []
            for b in range(NBUF):
                ds[b].wait()
                gs[b].wait()
                scs.append(
                    pltpu.async_copy(rows[b], acc.at[dbufs[b]],
                                     ssem.at[b], add=True))
            for cp in scs:
                cp.wait()

    @pl.when(c == 0)
    def _c0():
        _run_chunks(0, K0)

    @pl.when(c == 1)
    def _c1():
        _run_chunks(K0, K1)

    plsc.subcore_barrier()
    pltpu.sync_copy(acc.at[pl.ds(s * RPS, RPS)],
                    out_hbm.at[c, pl.ds(s * RPS, RPS)])


def _tc_mm_body(x_ref, w_ref, h_ref):
    h_ref[...] = jnp.dot(x_ref[...], w_ref[...],
                         preferred_element_type=jnp.float32)


_tc_mm = pl.pallas_call(
    _tc_mm_body,
    grid=(NB,),
    in_specs=[
        pl.BlockSpec((R, D), lambda i: (i, 0)),
        pl.BlockSpec((D, D), lambda i: (0, 0)),
    ],
    out_specs=pl.BlockSpec((R, D), lambda i: (i, 0)),
    out_shape=jax.ShapeDtypeStruct((N, D), jnp.float32),
)


def _tc1_body(deg_ref, h_ref, g_ref, dinv_ref):
    deg = deg_ref[0, :, 0:1] + deg_ref[1, :, 0:1] + 1.0
    dinv = lax.rsqrt(deg)
    g_ref[...] = h_ref[...] * dinv
    dinv_ref[...] = dinv


_tc1 = pl.pallas_call(
    _tc1_body,
    grid=(NB,),
    in_specs=[
        pl.BlockSpec((2, R, D), lambda i: (0, i, 0)),
        pl.BlockSpec((R, D), lambda i: (i, 0)),
    ],
    out_specs=[
        pl.BlockSpec((R, D), lambda i: (i, 0)),
        pl.BlockSpec((R, 1), lambda i: (i, 0)),
    ],
    out_shape=[
        jax.ShapeDtypeStruct((N, D), jnp.float32),
        jax.ShapeDtypeStruct((N, 1), jnp.float32),
    ],
)


def _tc_mid_body(sp_ref, g_ref, dinv_ref, b_ref, w_ref, gout_ref):
    ssum = sp_ref[0] + sp_ref[1]
    dinv = dinv_ref[...]
    f = jnp.maximum(dinv * (ssum + g_ref[...]) + b_ref[...], 0.0)
    gout_ref[...] = jnp.dot(
        f, w_ref[...], preferred_element_type=jnp.float32) * dinv


_tc_mid = pl.pallas_call(
    _tc_mid_body,
    grid=(NB,),
    in_specs=[
        pl.BlockSpec((2, R, D), lambda i: (0, i, 0)),
        pl.BlockSpec((R, D), lambda i: (i, 0)),
        pl.BlockSpec((R, 1), lambda i: (i, 0)),
        pl.BlockSpec((1, D), lambda i: (0, 0)),
        pl.BlockSpec((D, D), lambda i: (0, 0)),
    ],
    out_specs=pl.BlockSpec((R, D), lambda i: (i, 0)),
    out_shape=jax.ShapeDtypeStruct((N, D), jnp.float32),
)


def _tc_final_body(sp_ref, g_ref, dinv_ref, b_ref, batch_ref, wl_ref, bl_ref,
                   out_ref):
    ssum = sp_ref[0, :N, :] + sp_ref[1, :N, :]
    h = dinv_ref[...] * (ssum + g_ref[...]) + b_ref[...]
    gid = lax.broadcasted_iota(jnp.int32, (G, N), 0)
    oht = (batch_ref[...] == gid).astype(jnp.float32)
    sums = jnp.dot(oht, h, preferred_element_type=jnp.float32)
    counts = jnp.dot(oht, jnp.ones((N, 1), jnp.float32),
                     preferred_element_type=jnp.float32)
    pooled = sums / jnp.maximum(counts, 1.0)
    out_ref[...] = jnp.dot(
        pooled, wl_ref[...], preferred_element_type=jnp.float32) + bl_ref[...]


_tc_final = pl.pallas_call(
    _tc_final_body,
    out_shape=jax.ShapeDtypeStruct((G, 1), jnp.float32),
)


def kernel(x, edge_index, batch, W1, b1, W2, b2, W3, b3, Wl, bl):
    src = edge_index[0].astype(jnp.int32)
    dst = edge_index[1].astype(jnp.int32)
    pad = E_PAD - E
    src_r = jnp.pad(src, (0, pad)).reshape(NS, KT, CH)
    dst_r = jnp.pad(dst, (0, pad), constant_values=N).reshape(NS, KT, CH)
    batch_row = batch.astype(jnp.int32).reshape(1, N)
    b1r = b1.reshape(1, D)
    b2r = b2.reshape(1, D)
    b3r = b3.reshape(1, D)
    blr = bl.reshape(1, 1)

    degp = _deg_pass(dst_r)
    h1 = _tc_mm(x, W1)
    g1, dinv = _tc1(degp, h1)
    s1 = _edge_pass(g1, src_r, dst_r)
    g2 = _tc_mid(s1, g1, dinv, b1r, W2)
    s2 = _edge_pass(g2, src_r, dst_r)
    g3 = _tc_mid(s2, g2, dinv, b2r, W3)
    s3 = _edge_pass(g3, src_r, dst_r)
    return _tc_final(s3, g3, dinv, b3r, batch_row, Wl, blr)
